# baseline (device time: 66421 ns/iter reference)
import jax
import jax.numpy as jnp
from jax import lax
from jax.experimental import pallas as pl
from jax.experimental.pallas import tpu as pltpu

N_DEV = 4
B, SQ, SKV_SH, HQ, DH = 2, 256, 256, 16, 64
H_LOC = HQ // N_DEV
SKV = SKV_SH * N_DEV
D_MODEL = 512
ROWS = B * SQ
R_SH = ROWS // N_DEV
GQ = 32


def kernel(x, Wq, K_ext, V_ext, Wo):
    def body(x_ref, wq_ref, k_ref, v_ref, wo_ref, out_ref,
             ks_ref, vs_ref, kf_ref, vf_ref, pb_ref, rs_ref, red_ref, ag_ref,
             k_send, k_recv, v_send, v_recv,
             rs_send, rs_recv, ag_send, ag_recv, loc_sem):
        my = lax.axis_index("i")

        bsem = pltpu.get_barrier_semaphore()
        for d in range(1, N_DEV):
            pl.semaphore_signal(
                bsem, inc=1,
                device_id=((my + d) % N_DEV,),
                device_id_type=pl.DeviceIdType.MESH,
            )
        pl.semaphore_wait(bsem, N_DEV - 1)

        ks_ref[...] = k_ref[...].astype(jnp.bfloat16)
        vs_ref[...] = v_ref[...].astype(jnp.bfloat16)

        lck = pltpu.make_async_copy(
            ks_ref.at[:, :, pl.ds(H_LOC * my, H_LOC), :],
            kf_ref.at[my], loc_sem.at[0],
        )
        lcv = pltpu.make_async_copy(
            vs_ref.at[:, :, pl.ds(H_LOC * my, H_LOC), :],
            vf_ref.at[my], loc_sem.at[1],
        )
        lck.start()
        lcv.start()

        kv_rdmas = []
        for d in range(1, N_DEV):
            tgt = (my + d) % N_DEV
            rk = pltpu.make_async_remote_copy(
                src_ref=ks_ref.at[:, :, pl.ds(H_LOC * tgt, H_LOC), :],
                dst_ref=kf_ref.at[my],
                send_sem=k_send.at[d],
                recv_sem=k_recv.at[my],
                device_id=(tgt,),
                device_id_type=pl.DeviceIdType.MESH,
            )
            rv = pltpu.make_async_remote_copy(
                src_ref=vs_ref.at[:, :, pl.ds(H_LOC * tgt, H_LOC), :],
                dst_ref=vf_ref.at[my],
                send_sem=v_send.at[d],
                recv_sem=v_recv.at[my],
                device_id=(tgt,),
                device_id_type=pl.DeviceIdType.MESH,
            )
            rk.start()
            rv.start()
            kv_rdmas.append(rk)
            kv_rdmas.append(rv)

        qh = jnp.dot(
            x_ref[...].reshape(ROWS, D_MODEL).astype(jnp.bfloat16),
            wq_ref[...].astype(jnp.bfloat16),
            preferred_element_type=jnp.float32,
        )
        qh = (
            (qh * 0.125)
            .reshape(B, SQ, H_LOC, DH)
            .transpose(0, 2, 1, 3)
            .astype(jnp.bfloat16)
        )

        qi = lax.broadcasted_iota(jnp.int32, (SQ, SKV_SH), 0)
        kij = lax.broadcasted_iota(jnp.int32, (SQ, SKV_SH), 1)

        lck.wait()
        lcv.wait()

        def rs_send_chunk(tgt):
            dd = (tgt - my) % N_DEV

            @pl.when(my != tgt)
            def _():
                r = pltpu.make_async_remote_copy(
                    src_ref=pb_ref.at[pl.ds(R_SH * tgt, R_SH), :],
                    dst_ref=rs_ref.at[dd],
                    send_sem=rs_send.at[dd],
                    recv_sem=rs_recv.at[dd],
                    device_id=(tgt,),
                    device_id_type=pl.DeviceIdType.MESH,
                )
                r.start()

        wo_bf = wo_ref[...].astype(jnp.bfloat16)

        def wo_chunk(row_lo, n_rows, accs, lsums):
            parts = []
            for b in range(B):
                ctx = (
                    accs[b][:, row_lo:row_lo + n_rows, :]
                    / lsums[b][:, row_lo:row_lo + n_rows, None]
                )
                ctx = (
                    ctx.transpose(1, 0, 2)
                    .reshape(n_rows, H_LOC * DH)
                    .astype(jnp.bfloat16)
                )
                parts.append(jnp.dot(
                    ctx, wo_bf, preferred_element_type=jnp.float32,
                ))
            return jnp.concatenate(parts, axis=0)

        accs = [jnp.zeros((H_LOC, SQ, DH), jnp.float32) for _ in range(B)]
        lsums = [jnp.zeros((H_LOC, SQ), jnp.float32) for _ in range(B)]
        for s in range(N_DEV):
            @pl.when(my != s)
            def _():
                wk = pltpu.make_async_remote_copy(
                    src_ref=kf_ref.at[s],
                    dst_ref=kf_ref.at[s],
                    send_sem=k_send.at[0],
                    recv_sem=k_recv.at[s],
                    device_id=(s,),
                    device_id_type=pl.DeviceIdType.MESH,
                )
                wv = pltpu.make_async_remote_copy(
                    src_ref=vf_ref.at[s],
                    dst_ref=vf_ref.at[s],
                    send_sem=v_send.at[0],
                    recv_sem=v_recv.at[s],
                    device_id=(s,),
                    device_id_type=pl.DeviceIdType.MESH,
                )
                wk.wait_recv()
                wv.wait_recv()

            if s < 2:
                ki = kij + s * SKV_SH
                mask = (jnp.abs(qi - ki) <= 128) | (ki < GQ) | (qi < GQ)
            for b in range(B):
                kb = kf_ref[s, b]
                vb = vf_ref[s, b]
                if s < 2:
                    sc = lax.dot_general(
                        qh[b], kb,
                        dimension_numbers=(((2,), (2,)), ((0,), (1,))),
                        preferred_element_type=jnp.float32,
                    )
                    p = jnp.where(mask[None, :, :], jnp.exp(sc), 0.0)
                    lsums[b] = lsums[b] + jnp.sum(p, axis=-1)
                    accs[b] = accs[b] + lax.dot_general(
                        p.astype(jnp.bfloat16), vb,
                        dimension_numbers=(((2,), (0,)), ((0,), (1,))),
                        preferred_element_type=jnp.float32,
                    )
                else:
                    sc = lax.dot_general(
                        qh[b][:, :GQ, :], kb,
                        dimension_numbers=(((2,), (2,)), ((0,), (1,))),
                        preferred_element_type=jnp.float32,
                    )
                    p = jnp.exp(sc)
                    lsums[b] = lsums[b] + jnp.concatenate(
                        [jnp.sum(p, axis=-1),
                         jnp.zeros((H_LOC, SQ - GQ), jnp.float32)],
                        axis=1,
                    )
                    accs[b] = accs[b] + jnp.concatenate(
                        [lax.dot_general(
                            p.astype(jnp.bfloat16), vb,
                            dimension_numbers=(((2,), (0,)), ((0,), (1,))),
                            preferred_element_type=jnp.float32,
                        ),
                         jnp.zeros((H_LOC, SQ - GQ, DH), jnp.float32)],
                        axis=1,
                    )

            if s == 1:
                hi = wo_chunk(SQ - R_SH, R_SH, accs, lsums)
                pb_ref[pl.ds(R_SH, R_SH), :] = hi[:R_SH].astype(jnp.bfloat16)
                pb_ref[pl.ds(3 * R_SH, R_SH), :] = (
                    hi[R_SH:].astype(jnp.bfloat16)
                )
                rs_send_chunk(1)
                rs_send_chunk(3)

        lo = wo_chunk(0, R_SH, accs, lsums)
        pb_ref[pl.ds(0, R_SH), :] = lo[:R_SH].astype(jnp.bfloat16)
        pb_ref[pl.ds(2 * R_SH, R_SH), :] = lo[R_SH:].astype(jnp.bfloat16)
        rs_send_chunk(0)
        rs_send_chunk(2)

        for d in range(1, N_DEV):
            w = pltpu.make_async_remote_copy(
                src_ref=rs_ref.at[d],
                dst_ref=rs_ref.at[d],
                send_sem=rs_send.at[d],
                recv_sem=rs_recv.at[d],
                device_id=((my - d) % N_DEV,),
                device_id_type=pl.DeviceIdType.MESH,
            )
            w.wait_recv()
        red = pb_ref[pl.ds(R_SH * my, R_SH), :].astype(jnp.float32)
        for d in range(1, N_DEV):
            red = red + rs_ref[d].astype(jnp.float32)

        red_ref[...] = red.astype(jnp.bfloat16)
        ag_rdmas = []
        for d in range(1, N_DEV):
            tgt = (my + d) % N_DEV
            r = pltpu.make_async_remote_copy(
                src_ref=red_ref,
                dst_ref=ag_ref.at[d],
                send_sem=ag_send.at[d],
                recv_sem=ag_recv.at[d],
                device_id=(tgt,),
                device_id_type=pl.DeviceIdType.MESH,
            )
            r.start()
            ag_rdmas.append(r)

        out_ref[my // 2, pl.ds((my % 2) * R_SH, R_SH), :] = red

        for d in range(1, N_DEV):
            w = pltpu.make_async_remote_copy(
                src_ref=ag_ref.at[d],
                dst_ref=ag_ref.at[d],
                send_sem=ag_send.at[d],
                recv_sem=ag_recv.at[d],
                device_id=((my - d) % N_DEV,),
                device_id_type=pl.DeviceIdType.MESH,
            )
            w.wait_recv()
            src = (my - d) % N_DEV
            out_ref[src // 2, pl.ds((src % 2) * R_SH, R_SH), :] = (
                ag_ref[d].astype(jnp.float32)
            )

        for r in kv_rdmas + ag_rdmas:
            r.wait_send()
        for tgt in range(N_DEV):
            dd = (tgt - my) % N_DEV

            @pl.when(my != tgt)
            def _():
                w = pltpu.make_async_remote_copy(
                    src_ref=pb_ref.at[pl.ds(R_SH * tgt, R_SH), :],
                    dst_ref=rs_ref.at[dd],
                    send_sem=rs_send.at[dd],
                    recv_sem=rs_recv.at[dd],
                    device_id=(tgt,),
                    device_id_type=pl.DeviceIdType.MESH,
                )
                w.wait_send()

    return pl.pallas_call(
        body,
        out_shape=jax.ShapeDtypeStruct((B, SQ, D_MODEL), jnp.float32),
        in_specs=[pl.BlockSpec(memory_space=pltpu.VMEM)] * 5,
        out_specs=pl.BlockSpec(memory_space=pltpu.VMEM),
        scratch_shapes=[
            pltpu.VMEM((B, SKV_SH, HQ, DH), jnp.bfloat16),
            pltpu.VMEM((B, SKV_SH, HQ, DH), jnp.bfloat16),
            pltpu.VMEM((N_DEV, B, SKV_SH, H_LOC, DH), jnp.bfloat16),
            pltpu.VMEM((N_DEV, B, SKV_SH, H_LOC, DH), jnp.bfloat16),
            pltpu.VMEM((ROWS, D_MODEL), jnp.bfloat16),
            pltpu.VMEM((N_DEV, R_SH, D_MODEL), jnp.bfloat16),
            pltpu.VMEM((R_SH, D_MODEL), jnp.bfloat16),
            pltpu.VMEM((N_DEV, R_SH, D_MODEL), jnp.bfloat16),
            pltpu.SemaphoreType.DMA((N_DEV,)),
            pltpu.SemaphoreType.DMA((N_DEV,)),
            pltpu.SemaphoreType.DMA((N_DEV,)),
            pltpu.SemaphoreType.DMA((N_DEV,)),
            pltpu.SemaphoreType.DMA((N_DEV,)),
            pltpu.SemaphoreType.DMA((N_DEV,)),
            pltpu.SemaphoreType.DMA((N_DEV,)),
            pltpu.SemaphoreType.DMA((N_DEV,)),
            pltpu.SemaphoreType.DMA((2,)),
        ],
        compiler_params=pltpu.CompilerParams(collective_id=0),
    )(x, Wq, K_ext, V_ext, Wo)


# device time: 49287 ns/iter; 1.3476x vs baseline; 1.3476x over previous
import jax
import jax.numpy as jnp
from jax import lax
from jax.experimental import pallas as pl
from jax.experimental.pallas import tpu as pltpu

N_DEV = 4
B, SQ, SKV_SH, HQ, DH = 2, 256, 256, 16, 64
H_LOC = HQ // N_DEV
SKV = SKV_SH * N_DEV
D_MODEL = 512
BH = H_LOC * B
ROWS = B * SQ
R_SH = ROWS // N_DEV
GQ = 32
FROW = GQ + 1


def kernel(x, Wq, K_ext, V_ext, Wo):
    def body(x_ref, wq_ref, k_ref, v_ref, wo_ref, out_ref,
             ks_ref, vs_ref, kf_ref, vf_ref,
             qs_send, qs_ref, fps_ref, fpr_ref,
             pb_ref, rs_ref, red_ref, ag_ref,
             k_send, k_recv, v_send, v_recv,
             qsm_send, qsm_recv, fp_send, fp_recv,
             rs_send, rs_recv, ag_send, ag_recv, loc_sem):
        my = lax.axis_index("i")
        is_far_owner = my >= 2

        bsem = pltpu.get_barrier_semaphore()
        for d in range(1, N_DEV):
            pl.semaphore_signal(
                bsem, inc=1,
                device_id=((my + d) % N_DEV,),
                device_id_type=pl.DeviceIdType.MESH,
            )
        pl.semaphore_wait(bsem, N_DEV - 1)

        qfold = jnp.dot(
            x_ref[...].reshape(ROWS, D_MODEL).astype(jnp.bfloat16),
            wq_ref[...].astype(jnp.bfloat16),
            preferred_element_type=jnp.float32,
        )
        qfold = (
            (qfold * 0.125)
            .reshape(B, SQ, H_LOC, DH)
            .transpose(2, 0, 1, 3)
            .reshape(BH, SQ, DH)
            .astype(jnp.bfloat16)
        )

        qs_send[...] = qfold[:, :GQ, :]
        qs_local = pltpu.make_async_copy(qs_send, qs_ref.at[my], loc_sem.at[2])
        qs_local.start()
        for o in (2, 3):
            @pl.when(my != o)
            def _(o=o):
                r = pltpu.make_async_remote_copy(
                    src_ref=qs_send,
                    dst_ref=qs_ref.at[my],
                    send_sem=qsm_send.at[o],
                    recv_sem=qsm_recv.at[my],
                    device_id=(o,),
                    device_id_type=pl.DeviceIdType.MESH,
                )
                r.start()

        my_n = jnp.minimum(my, 1)
        lck = pltpu.make_async_copy(
            ks_ref.at[pl.ds(H_LOC * my_n, H_LOC)], kf_ref.at[my_n],
            loc_sem.at[0],
        )
        lcv = pltpu.make_async_copy(
            vs_ref.at[pl.ds(H_LOC * my_n, H_LOC)], vf_ref.at[my_n],
            loc_sem.at[1],
        )

        @pl.when(jnp.logical_not(is_far_owner))
        def _():
            ks_ref[...] = (
                k_ref[...].transpose(2, 0, 1, 3).astype(jnp.bfloat16)
            )
            vs_ref[...] = (
                v_ref[...].transpose(2, 0, 1, 3).astype(jnp.bfloat16)
            )
            lck.start()
            lcv.start()

        for s in (0, 1):
            for g in range(N_DEV):
                if g == s:
                    continue

                @pl.when(my == s)
                def _(s=s, g=g):
                    rk = pltpu.make_async_remote_copy(
                        src_ref=ks_ref.at[pl.ds(H_LOC * g, H_LOC)],
                        dst_ref=kf_ref.at[s],
                        send_sem=k_send.at[g],
                        recv_sem=k_recv.at[s],
                        device_id=(g,),
                        device_id_type=pl.DeviceIdType.MESH,
                    )
                    rv = pltpu.make_async_remote_copy(
                        src_ref=vs_ref.at[pl.ds(H_LOC * g, H_LOC)],
                        dst_ref=vf_ref.at[s],
                        send_sem=v_send.at[g],
                        recv_sem=v_recv.at[s],
                        device_id=(g,),
                        device_id_type=pl.DeviceIdType.MESH,
                    )
                    rk.start()
                    rv.start()

        qs_local.wait()
        for r in range(N_DEV):
            @pl.when(is_far_owner & (my != r))
            def _(r=r):
                w = pltpu.make_async_remote_copy(
                    src_ref=qs_ref.at[r],
                    dst_ref=qs_ref.at[r],
                    send_sem=qsm_send.at[0],
                    recv_sem=qsm_recv.at[r],
                    device_id=(r,),
                    device_id_type=pl.DeviceIdType.MESH,
                )
                w.wait_recv()

        def far_partials():
            msgs = []
            for r in range(N_DEV):
                qs_r = qs_ref[r].reshape(H_LOC, B, GQ, DH)
                packs = []
                for b in range(B):
                    kb = (
                        k_ref[b, :, H_LOC * r:H_LOC * (r + 1), :]
                        .transpose(1, 0, 2).astype(jnp.bfloat16)
                    )
                    vb = (
                        v_ref[b, :, H_LOC * r:H_LOC * (r + 1), :]
                        .transpose(1, 0, 2).astype(jnp.bfloat16)
                    )
                    sc = lax.dot_general(
                        qs_r[:, b], kb,
                        dimension_numbers=(((2,), (2,)), ((0,), (0,))),
                        preferred_element_type=jnp.float32,
                    )
                    p = jnp.exp(sc)
                    pv = lax.dot_general(
                        p.astype(jnp.bfloat16), vb,
                        dimension_numbers=(((2,), (1,)), ((0,), (0,))),
                        preferred_element_type=jnp.float32,
                    )
                    ps = jnp.sum(p, axis=-1)
                    ps_row = jnp.concatenate(
                        [ps[:, None, :],
                         jnp.zeros((H_LOC, 1, DH - GQ), jnp.float32)],
                        axis=2,
                    )
                    packs.append(jnp.concatenate([pv, ps_row], axis=1))
                msgs.append(jnp.stack(packs, axis=1))
            return jnp.stack(msgs, axis=0).astype(jnp.bfloat16)

        fps_ref[...] = lax.cond(
            is_far_owner,
            far_partials,
            lambda: jnp.zeros((N_DEV, H_LOC, B, FROW, DH), jnp.bfloat16),
        )

        my_f = jnp.maximum(my - 2, 0)
        fp_local = pltpu.make_async_copy(
            fps_ref.at[my], fpr_ref.at[my_f], loc_sem.at[3],
        )
        for r in range(N_DEV):
            @pl.when(is_far_owner & (my != r))
            def _(r=r):
                send = pltpu.make_async_remote_copy(
                    src_ref=fps_ref.at[r],
                    dst_ref=fpr_ref.at[my_f],
                    send_sem=fp_send.at[r],
                    recv_sem=fp_recv.at[my_f],
                    device_id=(r,),
                    device_id_type=pl.DeviceIdType.MESH,
                )
                send.start()

        @pl.when(is_far_owner)
        def _():
            fp_local.start()
            fp_local.wait()

        qi = lax.broadcasted_iota(jnp.int32, (SQ, SKV_SH), 0)
        kij = lax.broadcasted_iota(jnp.int32, (SQ, SKV_SH), 1)

        def rs_send_chunk(tgt):
            dd = (tgt - my) % N_DEV

            @pl.when(my != tgt)
            def _():
                r = pltpu.make_async_remote_copy(
                    src_ref=pb_ref.at[pl.ds(R_SH * tgt, R_SH), :],
                    dst_ref=rs_ref.at[dd],
                    send_sem=rs_send.at[dd],
                    recv_sem=rs_recv.at[dd],
                    device_id=(tgt,),
                    device_id_type=pl.DeviceIdType.MESH,
                )
                r.start()

        wo_bf = wo_ref[...].astype(jnp.bfloat16)

        def wo_chunk(row_lo, n_rows, acc, lsum):
            ctx = (
                acc[:, row_lo:row_lo + n_rows, :]
                / lsum[:, row_lo:row_lo + n_rows, None]
            )
            ctx = (
                ctx.reshape(H_LOC, B, n_rows, DH)
                .transpose(1, 2, 0, 3)
                .reshape(B * n_rows, H_LOC * DH)
                .astype(jnp.bfloat16)
            )
            return jnp.dot(ctx, wo_bf, preferred_element_type=jnp.float32)

        acc = jnp.zeros((BH, SQ, DH), jnp.float32)
        lsum = jnp.zeros((BH, SQ), jnp.float32)
        for s in (0, 1):
            @pl.when(my == s)
            def _():
                lck.wait()
                lcv.wait()

            @pl.when(my != s)
            def _(s=s):
                wk = pltpu.make_async_remote_copy(
                    src_ref=kf_ref.at[s],
                    dst_ref=kf_ref.at[s],
                    send_sem=k_send.at[0],
                    recv_sem=k_recv.at[s],
                    device_id=(s,),
                    device_id_type=pl.DeviceIdType.MESH,
                )
                wv = pltpu.make_async_remote_copy(
                    src_ref=vf_ref.at[s],
                    dst_ref=vf_ref.at[s],
                    send_sem=v_send.at[0],
                    recv_sem=v_recv.at[s],
                    device_id=(s,),
                    device_id_type=pl.DeviceIdType.MESH,
                )
                wk.wait_recv()
                wv.wait_recv()

            kb = kf_ref[s].reshape(BH, SKV_SH, DH)
            vb = vf_ref[s].reshape(BH, SKV_SH, DH)
            sc = lax.dot_general(
                qfold, kb,
                dimension_numbers=(((2,), (2,)), ((0,), (0,))),
                preferred_element_type=jnp.float32,
            )
            ki = kij + s * SKV_SH
            mask = (jnp.abs(qi - ki) <= 128) | (ki < GQ) | (qi < GQ)
            p = jnp.where(mask[None, :, :], jnp.exp(sc), 0.0)
            lsum = lsum + jnp.sum(p, axis=-1)
            acc = acc + lax.dot_general(
                p.astype(jnp.bfloat16), vb,
                dimension_numbers=(((2,), (1,)), ((0,), (0,))),
                preferred_element_type=jnp.float32,
            )

            if s == 1:
                hi = wo_chunk(SQ - R_SH, R_SH, acc, lsum)
                pb_ref[pl.ds(R_SH, R_SH), :] = hi[:R_SH].astype(jnp.bfloat16)
                pb_ref[pl.ds(3 * R_SH, R_SH), :] = (
                    hi[R_SH:].astype(jnp.bfloat16)
                )
                rs_send_chunk(1)
                rs_send_chunk(3)

        for o in (2, 3):
            @pl.when(my != o)
            def _(o=o):
                w = pltpu.make_async_remote_copy(
                    src_ref=fpr_ref.at[o - 2],
                    dst_ref=fpr_ref.at[o - 2],
                    send_sem=fp_send.at[0],
                    recv_sem=fp_recv.at[o - 2],
                    device_id=(o,),
                    device_id_type=pl.DeviceIdType.MESH,
                )
                w.wait_recv()
        far = fpr_ref[0].astype(jnp.float32) + fpr_ref[1].astype(jnp.float32)
        far_pv = far[:, :, :GQ, :].reshape(BH, GQ, DH)
        far_ps = far[:, :, GQ, :GQ].reshape(BH, GQ)
        acc = acc + jnp.concatenate(
            [far_pv, jnp.zeros((BH, SQ - GQ, DH), jnp.float32)], axis=1,
        )
        lsum = lsum + jnp.concatenate(
            [far_ps, jnp.zeros((BH, SQ - GQ), jnp.float32)], axis=1,
        )

        lo = wo_chunk(0, R_SH, acc, lsum)
        pb_ref[pl.ds(0, R_SH), :] = lo[:R_SH].astype(jnp.bfloat16)
        pb_ref[pl.ds(2 * R_SH, R_SH), :] = lo[R_SH:].astype(jnp.bfloat16)
        rs_send_chunk(0)
        rs_send_chunk(2)

        for d in range(1, N_DEV):
            w = pltpu.make_async_remote_copy(
                src_ref=rs_ref.at[d],
                dst_ref=rs_ref.at[d],
                send_sem=rs_send.at[d],
                recv_sem=rs_recv.at[d],
                device_id=((my - d) % N_DEV,),
                device_id_type=pl.DeviceIdType.MESH,
            )
            w.wait_recv()
        red = pb_ref[pl.ds(R_SH * my, R_SH), :].astype(jnp.float32)
        for d in range(1, N_DEV):
            red = red + rs_ref[d].astype(jnp.float32)

        red_ref[...] = red.astype(jnp.bfloat16)
        ag_rdmas = []
        for d in range(1, N_DEV):
            tgt = (my + d) % N_DEV
            r = pltpu.make_async_remote_copy(
                src_ref=red_ref,
                dst_ref=ag_ref.at[d],
                send_sem=ag_send.at[d],
                recv_sem=ag_recv.at[d],
                device_id=(tgt,),
                device_id_type=pl.DeviceIdType.MESH,
            )
            r.start()
            ag_rdmas.append(r)

        out_ref[my // 2, pl.ds((my % 2) * R_SH, R_SH), :] = red

        for d in range(1, N_DEV):
            w = pltpu.make_async_remote_copy(
                src_ref=ag_ref.at[d],
                dst_ref=ag_ref.at[d],
                send_sem=ag_send.at[d],
                recv_sem=ag_recv.at[d],
                device_id=((my - d) % N_DEV,),
                device_id_type=pl.DeviceIdType.MESH,
            )
            w.wait_recv()
            src = (my - d) % N_DEV
            out_ref[src // 2, pl.ds((src % 2) * R_SH, R_SH), :] = (
                ag_ref[d].astype(jnp.float32)
            )

        for r in ag_rdmas:
            r.wait_send()
        for s in (0, 1):
            for g in range(N_DEV):
                if g == s:
                    continue

                @pl.when(my == s)
                def _(s=s, g=g):
                    wk = pltpu.make_async_remote_copy(
                        src_ref=ks_ref.at[pl.ds(H_LOC * g, H_LOC)],
                        dst_ref=kf_ref.at[s],
                        send_sem=k_send.at[g],
                        recv_sem=k_recv.at[s],
                        device_id=(g,),
                        device_id_type=pl.DeviceIdType.MESH,
                    )
                    wv = pltpu.make_async_remote_copy(
                        src_ref=vs_ref.at[pl.ds(H_LOC * g, H_LOC)],
                        dst_ref=vf_ref.at[s],
                        send_sem=v_send.at[g],
                        recv_sem=v_recv.at[s],
                        device_id=(g,),
                        device_id_type=pl.DeviceIdType.MESH,
                    )
                    wk.wait_send()
                    wv.wait_send()
        for o in (2, 3):
            @pl.when(my != o)
            def _(o=o):
                w = pltpu.make_async_remote_copy(
                    src_ref=qs_send,
                    dst_ref=qs_ref.at[my],
                    send_sem=qsm_send.at[o],
                    recv_sem=qsm_recv.at[my],
                    device_id=(o,),
                    device_id_type=pl.DeviceIdType.MESH,
                )
                w.wait_send()
        for r in range(N_DEV):
            @pl.when(is_far_owner & (my != r))
            def _(r=r):
                w = pltpu.make_async_remote_copy(
                    src_ref=fps_ref.at[r],
                    dst_ref=fpr_ref.at[my_f],
                    send_sem=fp_send.at[r],
                    recv_sem=fp_recv.at[my_f],
                    device_id=(r,),
                    device_id_type=pl.DeviceIdType.MESH,
                )
                w.wait_send()
        for tgt in range(N_DEV):
            dd = (tgt - my) % N_DEV

            @pl.when(my != tgt)
            def _(tgt=tgt, dd=dd):
                w = pltpu.make_async_remote_copy(
                    src_ref=pb_ref.at[pl.ds(R_SH * tgt, R_SH), :],
                    dst_ref=rs_ref.at[dd],
                    send_sem=rs_send.at[dd],
                    recv_sem=rs_recv.at[dd],
                    device_id=(tgt,),
                    device_id_type=pl.DeviceIdType.MESH,
                )
                w.wait_send()

    return pl.pallas_call(
        body,
        out_shape=jax.ShapeDtypeStruct((B, SQ, D_MODEL), jnp.float32),
        in_specs=[pl.BlockSpec(memory_space=pltpu.VMEM)] * 5,
        out_specs=pl.BlockSpec(memory_space=pltpu.VMEM),
        scratch_shapes=[
            pltpu.VMEM((HQ, B, SKV_SH, DH), jnp.bfloat16),
            pltpu.VMEM((HQ, B, SKV_SH, DH), jnp.bfloat16),
            pltpu.VMEM((2, H_LOC, B, SKV_SH, DH), jnp.bfloat16),
            pltpu.VMEM((2, H_LOC, B, SKV_SH, DH), jnp.bfloat16),
            pltpu.VMEM((BH, GQ, DH), jnp.bfloat16),
            pltpu.VMEM((N_DEV, BH, GQ, DH), jnp.bfloat16),
            pltpu.VMEM((N_DEV, H_LOC, B, FROW, DH), jnp.bfloat16),
            pltpu.VMEM((2, H_LOC, B, FROW, DH), jnp.bfloat16),
            pltpu.VMEM((ROWS, D_MODEL), jnp.bfloat16),
            pltpu.VMEM((N_DEV, R_SH, D_MODEL), jnp.bfloat16),
            pltpu.VMEM((R_SH, D_MODEL), jnp.bfloat16),
            pltpu.VMEM((N_DEV, R_SH, D_MODEL), jnp.bfloat16),
            pltpu.SemaphoreType.DMA((N_DEV,)),
            pltpu.SemaphoreType.DMA((2,)),
            pltpu.SemaphoreType.DMA((N_DEV,)),
            pltpu.SemaphoreType.DMA((2,)),
            pltpu.SemaphoreType.DMA((N_DEV,)),
            pltpu.SemaphoreType.DMA((N_DEV,)),
            pltpu.SemaphoreType.DMA((N_DEV,)),
            pltpu.SemaphoreType.DMA((2,)),
            pltpu.SemaphoreType.DMA((N_DEV,)),
            pltpu.SemaphoreType.DMA((N_DEV,)),
            pltpu.SemaphoreType.DMA((N_DEV,)),
            pltpu.SemaphoreType.DMA((N_DEV,)),
            pltpu.SemaphoreType.DMA((4,)),
        ],
        compiler_params=pltpu.CompilerParams(collective_id=0),
    )(x, Wq, K_ext, V_ext, Wo)


# device time: 48811 ns/iter; 1.3608x vs baseline; 1.0098x over previous
import jax
import jax.numpy as jnp
from jax import lax
from jax.experimental import pallas as pl
from jax.experimental.pallas import tpu as pltpu

N_DEV = 4
B, SQ, SKV_SH, HQ, DH = 2, 256, 256, 16, 64
H_LOC = HQ // N_DEV
SKV = SKV_SH * N_DEV
D_MODEL = 512
BH = B * H_LOC
ROWS = B * SQ
R_SH = ROWS // N_DEV


def kernel(x, Wq, K_ext, V_ext, Wo):
    def body(x_ref, wq_ref, k_ref, v_ref, wo_ref, out_ref,
             kvs_ref, kvf_ref, pb_ref, rs_ref, red_ref, ag_ref,
             kv_send, kv_recv, rs_send, rs_recv, ag_send, ag_recv, loc_sem):
        my = lax.axis_index("i")

        bsem = pltpu.get_barrier_semaphore()
        for d in range(1, N_DEV):
            pl.semaphore_signal(
                bsem, inc=1,
                device_id=((my + d) % N_DEV,),
                device_id_type=pl.DeviceIdType.MESH,
            )
        pl.semaphore_wait(bsem, N_DEV - 1)

        kvs_ref[0] = k_ref[...].astype(jnp.bfloat16)
        kvs_ref[1] = v_ref[...].astype(jnp.bfloat16)

        lc = pltpu.make_async_copy(
            kvs_ref.at[:, :, :, pl.ds(H_LOC * my, H_LOC), :],
            kvf_ref.at[0],
            loc_sem.at[0],
        )
        lc.start()

        kv_rdmas = []
        for d in range(1, N_DEV):
            tgt = (my + d) % N_DEV
            r = pltpu.make_async_remote_copy(
                src_ref=kvs_ref.at[:, :, :, pl.ds(H_LOC * tgt, H_LOC), :],
                dst_ref=kvf_ref.at[d],
                send_sem=kv_send.at[d],
                recv_sem=kv_recv.at[d],
                device_id=(tgt,),
                device_id_type=pl.DeviceIdType.MESH,
            )
            r.start()
            kv_rdmas.append(r)

        qb = jnp.dot(
            x_ref[...].reshape(ROWS, D_MODEL).astype(jnp.bfloat16),
            wq_ref[...].astype(jnp.bfloat16),
            preferred_element_type=jnp.float32,
        )
        qb = (
            qb.reshape(B, SQ, H_LOC, DH)
            .transpose(0, 2, 1, 3)
            .reshape(BH, SQ, DH)
            .astype(jnp.bfloat16)
        )

        qi = lax.broadcasted_iota(jnp.int32, (SQ, SKV_SH), 0)
        kij = lax.broadcasted_iota(jnp.int32, (SQ, SKV_SH), 1)

        acc = jnp.zeros((BH, SQ, DH), jnp.float32)
        lsum = jnp.zeros((BH, SQ), jnp.float32)
        for d in range(N_DEV):
            if d == 0:
                lc.wait()
            else:
                w = pltpu.make_async_remote_copy(
                    src_ref=kvf_ref.at[d],
                    dst_ref=kvf_ref.at[d],
                    send_sem=kv_send.at[d],
                    recv_sem=kv_recv.at[d],
                    device_id=((my - d) % N_DEV,),
                    device_id_type=pl.DeviceIdType.MESH,
                )
                w.wait_recv()
            src = (my - d) % N_DEV
            kv = kvf_ref[d]
            kb = kv[0].transpose(0, 2, 1, 3).reshape(BH, SKV_SH, DH)
            vb = kv[1].transpose(0, 2, 1, 3).reshape(BH, SKV_SH, DH)
            s = lax.dot_general(
                qb, kb,
                dimension_numbers=(((2,), (2,)), ((0,), (0,))),
                preferred_element_type=jnp.float32,
            ) * 0.125
            ki = kij + src * SKV_SH
            mask = (jnp.abs(qi - ki) <= 128) | (ki < 32) | (qi < 32)
            p = jnp.where(mask[None, :, :], jnp.exp(s), 0.0)
            lsum = lsum + jnp.sum(p, axis=-1)
            acc = acc + lax.dot_general(
                p.astype(jnp.bfloat16), vb,
                dimension_numbers=(((2,), (1,)), ((0,), (0,))),
                preferred_element_type=jnp.float32,
            )

        ctx = acc / lsum[:, :, None]
        ctx = (
            ctx.reshape(B, H_LOC, SQ, DH)
            .transpose(0, 2, 1, 3)
            .reshape(ROWS, H_LOC * DH)
            .astype(jnp.bfloat16)
        )
        partial = jnp.dot(
            ctx, wo_ref[...].astype(jnp.bfloat16),
            preferred_element_type=jnp.float32,
        )
        pb_ref[...] = partial.astype(jnp.bfloat16)

        rs_rdmas = []
        for d in range(1, N_DEV):
            tgt = (my + d) % N_DEV
            r = pltpu.make_async_remote_copy(
                src_ref=pb_ref.at[pl.ds(R_SH * tgt, R_SH), :],
                dst_ref=rs_ref.at[d],
                send_sem=rs_send.at[d],
                recv_sem=rs_recv.at[d],
                device_id=(tgt,),
                device_id_type=pl.DeviceIdType.MESH,
            )
            r.start()
            rs_rdmas.append(r)
        for d in range(1, N_DEV):
            w = pltpu.make_async_remote_copy(
                src_ref=rs_ref.at[d],
                dst_ref=rs_ref.at[d],
                send_sem=rs_send.at[d],
                recv_sem=rs_recv.at[d],
                device_id=((my - d) % N_DEV,),
                device_id_type=pl.DeviceIdType.MESH,
            )
            w.wait_recv()
        red = pb_ref[pl.ds(R_SH * my, R_SH), :].astype(jnp.float32)
        for d in range(1, N_DEV):
            red = red + rs_ref[d].astype(jnp.float32)

        red_ref[...] = red.astype(jnp.bfloat16)
        ag_rdmas = []
        for d in range(1, N_DEV):
            tgt = (my + d) % N_DEV
            r = pltpu.make_async_remote_copy(
                src_ref=red_ref,
                dst_ref=ag_ref.at[d],
                send_sem=ag_send.at[d],
                recv_sem=ag_recv.at[d],
                device_id=(tgt,),
                device_id_type=pl.DeviceIdType.MESH,
            )
            r.start()
            ag_rdmas.append(r)

        my_b = my // 2
        my_off = (my % 2) * R_SH
        out_ref[my_b, pl.ds(my_off, R_SH), :] = red

        for d in range(1, N_DEV):
            w = pltpu.make_async_remote_copy(
                src_ref=ag_ref.at[d],
                dst_ref=ag_ref.at[d],
                send_sem=ag_send.at[d],
                recv_sem=ag_recv.at[d],
                device_id=((my - d) % N_DEV,),
                device_id_type=pl.DeviceIdType.MESH,
            )
            w.wait_recv()
            src = (my - d) % N_DEV
            out_ref[src // 2, pl.ds((src % 2) * R_SH, R_SH), :] = (
                ag_ref[d].astype(jnp.float32)
            )

        for r in kv_rdmas + rs_rdmas + ag_rdmas:
            r.wait_send()

    return pl.pallas_call(
        body,
        out_shape=jax.ShapeDtypeStruct((B, SQ, D_MODEL), jnp.float32),
        in_specs=[pl.BlockSpec(memory_space=pltpu.VMEM)] * 5,
        out_specs=pl.BlockSpec(memory_space=pltpu.VMEM),
        scratch_shapes=[
            pltpu.VMEM((2, B, SKV_SH, HQ, DH), jnp.bfloat16),
            pltpu.VMEM((N_DEV, 2, B, SKV_SH, H_LOC, DH), jnp.bfloat16),
            pltpu.VMEM((ROWS, D_MODEL), jnp.bfloat16),
            pltpu.VMEM((N_DEV, R_SH, D_MODEL), jnp.bfloat16),
            pltpu.VMEM((R_SH, D_MODEL), jnp.bfloat16),
            pltpu.VMEM((N_DEV, R_SH, D_MODEL), jnp.bfloat16),
            pltpu.SemaphoreType.DMA((N_DEV,)),
            pltpu.SemaphoreType.DMA((N_DEV,)),
            pltpu.SemaphoreType.DMA((N_DEV,)),
            pltpu.SemaphoreType.DMA((N_DEV,)),
            pltpu.SemaphoreType.DMA((N_DEV,)),
            pltpu.SemaphoreType.DMA((N_DEV,)),
            pltpu.SemaphoreType.DMA((2,)),
        ],
        compiler_params=pltpu.CompilerParams(collective_id=0),
    )(x, Wq, K_ext, V_ext, Wo)
